# Initial kernel scaffold; baseline (speedup 1.0000x reference)
import jax
import jax.numpy as jnp
from jax.experimental import pallas as pl


def kernel(x, edge_index, W_enc, b_enc, W_dec, b_dec):
    raise NotImplementedError("write your pallas kernel here")


# speedup vs baseline: 10.7502x; 10.7502x over previous
import functools

import jax
import jax.numpy as jnp
from jax import lax
from jax.experimental import pallas as pl
from jax.experimental.pallas import tpu as pltpu
from jax.experimental.pallas import tpu_sc as plsc

NC = 2
NS = 16
LANES = 16

N_PAD = 10240


def _sc_mesh():
    return plsc.VectorSubcoreMesh(
        core_axis_name="c", subcore_axis_name="s", num_cores=NC, num_subcores=NS
    )


def _sc_deg(dst):
    E = dst.shape[0]
    NW = NC * NS
    per_tile = E // NW
    n_chunks = per_tile // 128
    tail = per_tile - n_chunks * 128

    @functools.partial(
        pl.kernel,
        out_type=jax.ShapeDtypeStruct((NW, N_PAD), jnp.float32),
        mesh=_sc_mesh(),
        scratch_types=[
            pltpu.VMEM((N_PAD,), jnp.float32),
            pltpu.VMEM((128,), jnp.int32),
        ],
        compiler_params=pltpu.CompilerParams(needs_layout_passes=False),
    )
    def deg_kernel(dst_hbm, out_hbm, degv, idxv):
        c = lax.axis_index("c")
        s = lax.axis_index("s")
        wid = s * NC + c
        base = wid * per_tile

        zeros = jnp.zeros((LANES,), jnp.float32)

        def zbody(i, carry):
            degv[pl.ds(i * LANES, LANES)] = zeros
            return carry

        lax.fori_loop(0, N_PAD // LANES, zbody, 0)

        ones = jnp.ones((LANES,), jnp.float32)

        def cbody(k, carry):
            pltpu.sync_copy(dst_hbm.at[pl.ds(base + k * 128, 128)], idxv)
            for j in range(128 // LANES):
                idx = idxv[pl.ds(j * LANES, LANES)]
                plsc.addupdate_scatter(degv, [idx], ones)
            return carry

        lax.fori_loop(0, n_chunks, cbody, 0)

        if tail:
            pltpu.sync_copy(
                dst_hbm.at[pl.ds(base + per_tile - LANES, LANES)], idxv.at[pl.ds(0, LANES)]
            )
            idx = idxv[pl.ds(0, LANES)]
            mask = lax.broadcasted_iota(jnp.int32, (LANES,), 0) >= (LANES - tail)
            plsc.addupdate_scatter(degv, [idx], ones, mask=mask)

        pltpu.sync_copy(degv, out_hbm.at[wid])

    return deg_kernel(dst)


def _sc_agg(t_a, t_b, seed_a, seed_b, src, dst, edges_per_core, core_stride):
    N, Ch = t_a.shape
    per_tile = edges_per_core // NS
    n_chunks = per_tile // 128
    rem = per_tile - n_chunks * 128
    dup = 16 - rem if rem else 0
    rows_per_tile = N // NS

    @functools.partial(
        pl.kernel,
        out_type=(
            jax.ShapeDtypeStruct((N, Ch), jnp.float32),
            jax.ShapeDtypeStruct((N, Ch), jnp.float32),
        ),
        mesh=_sc_mesh(),
        scratch_types=[
            pltpu.VMEM_SHARED((N, Ch), jnp.float32),
            pltpu.VMEM((128,), jnp.int32),
            pltpu.VMEM((128,), jnp.int32),
            pltpu.VMEM((16,), jnp.int32),
            pltpu.VMEM((16,), jnp.int32),
            pltpu.VMEM((128, Ch), jnp.float32),
            pltpu.VMEM((16, Ch), jnp.float32),
            pltpu.SemaphoreType.DMA,
        ],
    )
    def agg_kernel(
        ta_hbm, tb_hbm, sa_hbm, sb_hbm, src_hbm, dst_hbm, outa_hbm, outb_hbm,
        acc, idxs, idxd, idxs_t, idxd_t, rows, rows_t, sem,
    ):
        c = lax.axis_index("c")
        s = lax.axis_index("s")
        rbase = s * rows_per_tile

        def run(t_hbm, seed_hbm, out_hbm):
            base = c * core_stride + s * per_tile
            pltpu.sync_copy(
                seed_hbm.at[pl.ds(rbase, rows_per_tile)],
                acc.at[pl.ds(rbase, rows_per_tile)],
            )
            plsc.subcore_barrier()

            def cbody(k, carry):
                off = base + k * 128
                pltpu.sync_copy(src_hbm.at[pl.ds(off, 128)], idxs)
                pltpu.sync_copy(dst_hbm.at[pl.ds(off, 128)], idxd)
                pltpu.async_copy(t_hbm.at[idxs], rows, sem).wait()
                pltpu.sync_copy(rows, acc.at[idxd], add=True)
                return carry

            lax.fori_loop(0, n_chunks, cbody, 0)

            if rem:
                off = base + per_tile - 16
                pltpu.sync_copy(src_hbm.at[pl.ds(off, 16)], idxs_t)
                pltpu.sync_copy(dst_hbm.at[pl.ds(off, 16)], idxd_t)
                if dup:
                    lane = lax.broadcasted_iota(jnp.int32, (16,), 0)
                    idxd_t[...] = jnp.where(lane < dup, 10000 + lane, idxd_t[...])
                pltpu.async_copy(t_hbm.at[idxs_t], rows_t, sem).wait()
                pltpu.sync_copy(rows_t, acc.at[idxd_t], add=True)

            plsc.subcore_barrier()
            pltpu.sync_copy(
                acc.at[pl.ds(rbase, rows_per_tile)],
                out_hbm.at[pl.ds(rbase, rows_per_tile)],
            )

        @pl.when(c == 0)
        def _():
            run(ta_hbm, sa_hbm, outa_hbm)

        @pl.when(c == 1)
        def _():
            run(tb_hbm, sb_hbm, outb_hbm)

    return agg_kernel(t_a, t_b, seed_a, seed_b, src, dst)


def _tc_dinv(parts):
    NW, NP = parts.shape
    B = 128

    def body(p_ref, o_ref):
        ssum = jnp.sum(p_ref[...], axis=0, keepdims=True)
        d = lax.rsqrt(1.0 + ssum)
        i = lax.broadcasted_iota(jnp.int32, (B, B), 0)
        j = lax.broadcasted_iota(jnp.int32, (B, B), 1)
        eye = (i == j).astype(jnp.float32)
        o_ref[...] = lax.dot_general(
            eye, d, (((1,), (1,)), ((), ())), preferred_element_type=jnp.float32
        )

    return pl.pallas_call(
        body,
        grid=(NP // B,),
        in_specs=[pl.BlockSpec((NW, B), lambda m: (0, m))],
        out_specs=pl.BlockSpec((B, 1), lambda m: (m, 0)),
        out_shape=jax.ShapeDtypeStruct((NP, 1), jnp.float32),
    )(parts)


def _tc_enc(x, W_enc, dinv):
    N, IN = x.shape
    HID = W_enc.shape[1]
    BM = 2000

    def body(x_ref, w_ref, d_ref, o_ref):
        xw = jnp.dot(x_ref[...], w_ref[...], preferred_element_type=jnp.float32)
        o_ref[...] = xw * d_ref[...]

    return pl.pallas_call(
        body,
        grid=(N // BM,),
        in_specs=[
            pl.BlockSpec((BM, IN), lambda m: (m, 0)),
            pl.BlockSpec((IN, HID), lambda m: (0, 0)),
            pl.BlockSpec((BM, 1), lambda m: (m, 0)),
        ],
        out_specs=pl.BlockSpec((BM, HID), lambda m: (m, 0)),
        out_shape=jax.ShapeDtypeStruct((N_PAD, HID), jnp.float32),
    )(x, W_enc, dinv)


def _tc_mid(g_a, g_b, dinv, b_enc, W_dec):
    _, HID = g_a.shape
    N = 10000
    OUT = W_dec.shape[1]
    O2 = OUT // 2
    BM = 2000

    def body(ga_ref, gb_ref, d_ref, b_ref, w_ref, oa_ref, ob_ref):
        agg = ga_ref[...] + gb_ref[...]
        d = d_ref[...]
        h = jnp.maximum(agg * d + b_ref[...], 0.0)
        hw = jnp.dot(h, w_ref[...], preferred_element_type=jnp.float32)
        t = hw * d
        oa_ref[...] = t[:, :O2]
        ob_ref[...] = t[:, O2:]

    return pl.pallas_call(
        body,
        grid=(N // BM,),
        in_specs=[
            pl.BlockSpec((BM, HID), lambda m: (m, 0)),
            pl.BlockSpec((BM, HID), lambda m: (m, 0)),
            pl.BlockSpec((BM, 1), lambda m: (m, 0)),
            pl.BlockSpec((1, HID), lambda m: (0, 0)),
            pl.BlockSpec((HID, OUT), lambda m: (0, 0)),
        ],
        out_specs=[
            pl.BlockSpec((BM, O2), lambda m: (m, 0)),
            pl.BlockSpec((BM, O2), lambda m: (m, 0)),
        ],
        out_shape=[
            jax.ShapeDtypeStruct((N_PAD, O2), jnp.float32),
            jax.ShapeDtypeStruct((N_PAD, O2), jnp.float32),
        ],
    )(g_a, g_b, dinv, b_enc, W_dec)


def _tc_out(g_a, g_b, dinv, b_dec):
    _, O2 = g_a.shape
    N = 10000
    OUT = 2 * O2
    BM = 2000

    def body(ga_ref, gb_ref, d_ref, b_ref, o_ref):
        agg = jnp.concatenate([ga_ref[...], gb_ref[...]], axis=1)
        o_ref[...] = agg * d_ref[...] + b_ref[...]

    return pl.pallas_call(
        body,
        grid=(N // BM,),
        in_specs=[
            pl.BlockSpec((BM, O2), lambda m: (m, 0)),
            pl.BlockSpec((BM, O2), lambda m: (m, 0)),
            pl.BlockSpec((BM, 1), lambda m: (m, 0)),
            pl.BlockSpec((1, OUT), lambda m: (0, 0)),
        ],
        out_specs=pl.BlockSpec((BM, OUT), lambda m: (m, 0)),
        out_shape=jax.ShapeDtypeStruct((N, OUT), jnp.float32),
    )(g_a, g_b, dinv, b_dec)


def kernel(x, edge_index, W_enc, b_enc, W_dec, b_dec):
    src = edge_index[0].astype(jnp.int32)
    dst = edge_index[1].astype(jnp.int32)

    E = src.shape[0]

    parts = _sc_deg(dst)
    dinv = _tc_dinv(parts)
    t1 = _tc_enc(x, W_enc, dinv)
    zeros = jnp.zeros_like(t1)
    g1a, g1b = _sc_agg(t1, t1, t1, zeros, src, dst, E // 2, E // 2)
    t2a, t2b = _tc_mid(g1a, g1b, dinv, b_enc.reshape(1, -1), W_dec)
    g2a, g2b = _sc_agg(t2a, t2b, t2a, t2b, src, dst, E, 0)
    return _tc_out(g2a, g2b, dinv, b_dec.reshape(1, -1))


# speedup vs baseline: 16.5674x; 1.5411x over previous
import functools

import jax
import jax.numpy as jnp
from jax import lax
from jax.experimental import pallas as pl
from jax.experimental.pallas import tpu as pltpu
from jax.experimental.pallas import tpu_sc as plsc

NC = 2
NS = 16
LANES = 16

N_PAD = 10240


def _sc_mesh():
    return plsc.VectorSubcoreMesh(
        core_axis_name="c", subcore_axis_name="s", num_cores=NC, num_subcores=NS
    )


def _sc_deg(dst):
    E = dst.shape[0]
    NW = NC * NS
    per_tile = E // NW
    n_chunks = per_tile // 128
    tail = per_tile - n_chunks * 128

    @functools.partial(
        pl.kernel,
        out_type=jax.ShapeDtypeStruct((NW, N_PAD), jnp.float32),
        mesh=_sc_mesh(),
        scratch_types=[
            pltpu.VMEM((N_PAD,), jnp.float32),
            pltpu.VMEM((128,), jnp.int32),
        ],
        compiler_params=pltpu.CompilerParams(needs_layout_passes=False),
    )
    def deg_kernel(dst_hbm, out_hbm, degv, idxv):
        c = lax.axis_index("c")
        s = lax.axis_index("s")
        wid = s * NC + c
        base = wid * per_tile

        zeros = jnp.zeros((LANES,), jnp.float32)

        def zbody(i, carry):
            degv[pl.ds(i * LANES, LANES)] = zeros
            return carry

        lax.fori_loop(0, N_PAD // LANES, zbody, 0)

        ones = jnp.ones((LANES,), jnp.float32)

        def cbody(k, carry):
            pltpu.sync_copy(dst_hbm.at[pl.ds(base + k * 128, 128)], idxv)
            for j in range(128 // LANES):
                idx = idxv[pl.ds(j * LANES, LANES)]
                plsc.addupdate_scatter(degv, [idx], ones)
            return carry

        lax.fori_loop(0, n_chunks, cbody, 0)

        if tail:
            pltpu.sync_copy(
                dst_hbm.at[pl.ds(base + per_tile - LANES, LANES)], idxv.at[pl.ds(0, LANES)]
            )
            idx = idxv[pl.ds(0, LANES)]
            mask = lax.broadcasted_iota(jnp.int32, (LANES,), 0) >= (LANES - tail)
            plsc.addupdate_scatter(degv, [idx], ones, mask=mask)

        pltpu.sync_copy(degv, out_hbm.at[wid])

    return deg_kernel(dst)


def _sc_agg(t_a, t_b, seed_a, seed_b, src, dst, edges_per_core, core_stride):
    N, Ch = t_a.shape
    per_tile = edges_per_core // NS
    n_chunks = per_tile // 128
    rem = per_tile - n_chunks * 128
    dup = 16 - rem if rem else 0
    rows_per_tile = N // NS

    @functools.partial(
        pl.kernel,
        out_type=(
            jax.ShapeDtypeStruct((N, Ch), jnp.float32),
            jax.ShapeDtypeStruct((N, Ch), jnp.float32),
        ),
        mesh=_sc_mesh(),
        scratch_types=[
            pltpu.VMEM_SHARED((N, Ch), jnp.float32),
            pltpu.VMEM((128,), jnp.int32),
            pltpu.VMEM((128,), jnp.int32),
            pltpu.VMEM((128,), jnp.int32),
            pltpu.VMEM((128,), jnp.int32),
            pltpu.VMEM((16,), jnp.int32),
            pltpu.VMEM((16,), jnp.int32),
            pltpu.VMEM((128, Ch), jnp.float32),
            pltpu.VMEM((128, Ch), jnp.float32),
            pltpu.VMEM((16, Ch), jnp.float32),
            pltpu.SemaphoreType.DMA,
            pltpu.SemaphoreType.DMA,
            pltpu.SemaphoreType.DMA,
            pltpu.SemaphoreType.DMA,
            pltpu.SemaphoreType.DMA,
            pltpu.SemaphoreType.DMA,
            pltpu.SemaphoreType.DMA,
            pltpu.SemaphoreType.DMA,
            pltpu.SemaphoreType.DMA,
        ],
    )
    def agg_kernel(
        ta_hbm, tb_hbm, sa_hbm, sb_hbm, src_hbm, dst_hbm, outa_hbm, outb_hbm,
        acc, is0, id0, is1, id1, idxs_t, idxd_t, rows0, rows1, rows_t,
        xsem0, ysem0, gsem0, ssem0, xsem1, ysem1, gsem1, ssem1, tsem,
    ):
        c = lax.axis_index("c")
        s = lax.axis_index("s")
        rbase = s * rows_per_tile

        def run(t_hbm, seed_hbm, out_hbm):
            base = c * core_stride + s * per_tile
            slots = ((is0, id0, rows0, xsem0, ysem0, gsem0, ssem0),
                     (is1, id1, rows1, xsem1, ysem1, gsem1, ssem1))

            def issue_idx(k, sl):
                isb, idb, _, xsem, ysem, _, _ = sl
                off = base + k * 128
                pltpu.async_copy(src_hbm.at[pl.ds(off, 128)], isb, xsem)
                pltpu.async_copy(dst_hbm.at[pl.ds(off, 128)], idb, ysem)

            def wait_idx(sl):
                isb, idb, _, xsem, ysem, _, _ = sl
                pltpu.make_async_copy(src_hbm.at[pl.ds(base, 128)], isb, xsem).wait()
                pltpu.make_async_copy(dst_hbm.at[pl.ds(base, 128)], idb, ysem).wait()

            def issue_gather(sl):
                isb, _, rws, _, _, gsem, _ = sl
                pltpu.async_copy(t_hbm.at[isb], rws, gsem)

            def wait_gather(sl):
                isb, _, rws, _, _, gsem, _ = sl
                pltpu.make_async_copy(t_hbm.at[isb], rws, gsem).wait()

            def issue_scatter(sl):
                _, idb, rws, _, _, _, ssem = sl
                pltpu.async_copy(rws, acc.at[idb], ssem, add=True)

            def wait_scatter(sl):
                _, idb, rws, _, _, _, ssem = sl
                pltpu.make_async_copy(rws, acc.at[idb], ssem).wait()

            issue_idx(0, slots[0])
            issue_idx(1, slots[1])
            pltpu.sync_copy(
                seed_hbm.at[pl.ds(rbase, rows_per_tile)],
                acc.at[pl.ds(rbase, rows_per_tile)],
            )
            wait_idx(slots[0])
            issue_gather(slots[0])
            plsc.subcore_barrier()

            def pair_body(g, carry):
                k0 = 2 * g
                s0, s1 = slots
                wait_gather(s0)
                issue_scatter(s0)
                wait_idx(s1)
                issue_gather(s1)
                wait_scatter(s0)

                @pl.when(k0 + 2 < n_chunks)
                def _():
                    issue_idx(k0 + 2, s0)

                wait_gather(s1)
                issue_scatter(s1)

                @pl.when(k0 + 2 < n_chunks)
                def _():
                    wait_idx(s0)
                    issue_gather(s0)

                wait_scatter(s1)

                @pl.when(k0 + 3 < n_chunks)
                def _():
                    issue_idx(k0 + 3, s1)

                return carry

            lax.fori_loop(0, n_chunks // 2, pair_body, 0)

            if n_chunks % 2:
                wait_gather(slots[0])
                issue_scatter(slots[0])
                wait_scatter(slots[0])

            if rem:
                off = base + per_tile - 16
                pltpu.sync_copy(src_hbm.at[pl.ds(off, 16)], idxs_t)
                pltpu.sync_copy(dst_hbm.at[pl.ds(off, 16)], idxd_t)
                if dup:
                    lane = lax.broadcasted_iota(jnp.int32, (16,), 0)
                    idxd_t[...] = jnp.where(lane < dup, 10000 + lane, idxd_t[...])
                pltpu.async_copy(t_hbm.at[idxs_t], rows_t, tsem).wait()
                pltpu.sync_copy(rows_t, acc.at[idxd_t], add=True)

            plsc.subcore_barrier()
            pltpu.sync_copy(
                acc.at[pl.ds(rbase, rows_per_tile)],
                out_hbm.at[pl.ds(rbase, rows_per_tile)],
            )

        @pl.when(c == 0)
        def _():
            run(ta_hbm, sa_hbm, outa_hbm)

        @pl.when(c == 1)
        def _():
            run(tb_hbm, sb_hbm, outb_hbm)

    return agg_kernel(t_a, t_b, seed_a, seed_b, src, dst)


def _tc_dinv(parts):
    NW, NP = parts.shape
    B = 128

    def body(p_ref, o_ref):
        ssum = jnp.sum(p_ref[...], axis=0, keepdims=True)
        d = lax.rsqrt(1.0 + ssum)
        i = lax.broadcasted_iota(jnp.int32, (B, B), 0)
        j = lax.broadcasted_iota(jnp.int32, (B, B), 1)
        eye = (i == j).astype(jnp.float32)
        o_ref[...] = lax.dot_general(
            eye, d, (((1,), (1,)), ((), ())), preferred_element_type=jnp.float32
        )

    return pl.pallas_call(
        body,
        grid=(NP // B,),
        in_specs=[pl.BlockSpec((NW, B), lambda m: (0, m))],
        out_specs=pl.BlockSpec((B, 1), lambda m: (m, 0)),
        out_shape=jax.ShapeDtypeStruct((NP, 1), jnp.float32),
    )(parts)


def _tc_enc(x, W_enc, dinv):
    N, IN = x.shape
    HID = W_enc.shape[1]
    BM = 2000

    def body(x_ref, w_ref, d_ref, o_ref):
        xw = jnp.dot(x_ref[...], w_ref[...], preferred_element_type=jnp.float32)
        o_ref[...] = xw * d_ref[...]

    return pl.pallas_call(
        body,
        grid=(N // BM,),
        in_specs=[
            pl.BlockSpec((BM, IN), lambda m: (m, 0)),
            pl.BlockSpec((IN, HID), lambda m: (0, 0)),
            pl.BlockSpec((BM, 1), lambda m: (m, 0)),
        ],
        out_specs=pl.BlockSpec((BM, HID), lambda m: (m, 0)),
        out_shape=jax.ShapeDtypeStruct((N_PAD, HID), jnp.float32),
    )(x, W_enc, dinv)


def _tc_mid(g_a, g_b, dinv, b_enc, W_dec):
    _, HID = g_a.shape
    N = 10000
    OUT = W_dec.shape[1]
    O2 = OUT // 2
    BM = 2000

    def body(ga_ref, gb_ref, d_ref, b_ref, w_ref, oa_ref, ob_ref):
        agg = ga_ref[...] + gb_ref[...]
        d = d_ref[...]
        h = jnp.maximum(agg * d + b_ref[...], 0.0)
        hw = jnp.dot(h, w_ref[...], preferred_element_type=jnp.float32)
        t = hw * d
        oa_ref[...] = t[:, :O2]
        ob_ref[...] = t[:, O2:]

    return pl.pallas_call(
        body,
        grid=(N // BM,),
        in_specs=[
            pl.BlockSpec((BM, HID), lambda m: (m, 0)),
            pl.BlockSpec((BM, HID), lambda m: (m, 0)),
            pl.BlockSpec((BM, 1), lambda m: (m, 0)),
            pl.BlockSpec((1, HID), lambda m: (0, 0)),
            pl.BlockSpec((HID, OUT), lambda m: (0, 0)),
        ],
        out_specs=[
            pl.BlockSpec((BM, O2), lambda m: (m, 0)),
            pl.BlockSpec((BM, O2), lambda m: (m, 0)),
        ],
        out_shape=[
            jax.ShapeDtypeStruct((N_PAD, O2), jnp.float32),
            jax.ShapeDtypeStruct((N_PAD, O2), jnp.float32),
        ],
    )(g_a, g_b, dinv, b_enc, W_dec)


def _tc_out(g_a, g_b, dinv, b_dec):
    _, O2 = g_a.shape
    N = 10000
    OUT = 2 * O2
    BM = 2000

    def body(ga_ref, gb_ref, d_ref, b_ref, o_ref):
        agg = jnp.concatenate([ga_ref[...], gb_ref[...]], axis=1)
        o_ref[...] = agg * d_ref[...] + b_ref[...]

    return pl.pallas_call(
        body,
        grid=(N // BM,),
        in_specs=[
            pl.BlockSpec((BM, O2), lambda m: (m, 0)),
            pl.BlockSpec((BM, O2), lambda m: (m, 0)),
            pl.BlockSpec((BM, 1), lambda m: (m, 0)),
            pl.BlockSpec((1, OUT), lambda m: (0, 0)),
        ],
        out_specs=pl.BlockSpec((BM, OUT), lambda m: (m, 0)),
        out_shape=jax.ShapeDtypeStruct((N, OUT), jnp.float32),
    )(g_a, g_b, dinv, b_dec)


def kernel(x, edge_index, W_enc, b_enc, W_dec, b_dec):
    src = edge_index[0].astype(jnp.int32)
    dst = edge_index[1].astype(jnp.int32)

    E = src.shape[0]

    parts = _sc_deg(dst)
    dinv = _tc_dinv(parts)
    t1 = _tc_enc(x, W_enc, dinv)
    zeros = jnp.zeros_like(t1)
    g1a, g1b = _sc_agg(t1, t1, t1, zeros, src, dst, E // 2, E // 2)
    t2a, t2b = _tc_mid(g1a, g1b, dinv, b_enc.reshape(1, -1), W_dec)
    g2a, g2b = _sc_agg(t2a, t2b, t2a, t2b, src, dst, E, 0)
    return _tc_out(g2a, g2b, dinv, b_dec.reshape(1, -1))


# speedup vs baseline: 19.0230x; 1.1482x over previous
import functools

import jax
import jax.numpy as jnp
from jax import lax
from jax.experimental import pallas as pl
from jax.experimental.pallas import tpu as pltpu
from jax.experimental.pallas import tpu_sc as plsc

NC = 2
NS = 16
LANES = 16

N_PAD = 10240


def _sc_mesh():
    return plsc.VectorSubcoreMesh(
        core_axis_name="c", subcore_axis_name="s", num_cores=NC, num_subcores=NS
    )


def _sc_deg(dst):
    E = dst.shape[0]
    NW = NC * NS
    per_tile = E // NW
    n_chunks = per_tile // 128
    tail = per_tile - n_chunks * 128

    @functools.partial(
        pl.kernel,
        out_type=jax.ShapeDtypeStruct((NW, N_PAD), jnp.float32),
        mesh=_sc_mesh(),
        scratch_types=[
            pltpu.VMEM((N_PAD,), jnp.float32),
            pltpu.VMEM((128,), jnp.int32),
        ],
        compiler_params=pltpu.CompilerParams(needs_layout_passes=False),
    )
    def deg_kernel(dst_hbm, out_hbm, degv, idxv):
        c = lax.axis_index("c")
        s = lax.axis_index("s")
        wid = s * NC + c
        base = wid * per_tile

        zeros = jnp.zeros((LANES,), jnp.float32)

        def zbody(i, carry):
            degv[pl.ds(i * LANES, LANES)] = zeros
            return carry

        lax.fori_loop(0, N_PAD // LANES, zbody, 0)

        ones = jnp.ones((LANES,), jnp.float32)

        def cbody(k, carry):
            pltpu.sync_copy(dst_hbm.at[pl.ds(base + k * 128, 128)], idxv)
            for j in range(128 // LANES):
                idx = idxv[pl.ds(j * LANES, LANES)]
                plsc.addupdate_scatter(degv, [idx], ones)
            return carry

        lax.fori_loop(0, n_chunks, cbody, 0)

        if tail:
            pltpu.sync_copy(
                dst_hbm.at[pl.ds(base + per_tile - LANES, LANES)], idxv.at[pl.ds(0, LANES)]
            )
            idx = idxv[pl.ds(0, LANES)]
            mask = lax.broadcasted_iota(jnp.int32, (LANES,), 0) >= (LANES - tail)
            plsc.addupdate_scatter(degv, [idx], ones, mask=mask)

        pltpu.sync_copy(degv, out_hbm.at[wid])

    return deg_kernel(dst)


def _sc_agg(t_a, t_b, seed_a, seed_b, src, dst, edges_per_core, core_stride):
    N, Ch = t_a.shape
    per_tile = edges_per_core // NS
    n_chunks = per_tile // 128
    rem = per_tile - n_chunks * 128
    dup = 16 - rem if rem else 0
    rows_per_tile = N // NS

    @functools.partial(
        pl.kernel,
        out_type=(
            jax.ShapeDtypeStruct((N, Ch), jnp.float32),
            jax.ShapeDtypeStruct((N, Ch), jnp.float32),
        ),
        mesh=_sc_mesh(),
        scratch_types=[
            pltpu.VMEM_SHARED((N, Ch), jnp.float32),
            pltpu.VMEM((128,), jnp.int32),
            pltpu.VMEM((128,), jnp.int32),
            pltpu.VMEM((128,), jnp.int32),
            pltpu.VMEM((128,), jnp.int32),
            pltpu.VMEM((16,), jnp.int32),
            pltpu.VMEM((16,), jnp.int32),
            pltpu.VMEM((128, Ch), jnp.float32),
            pltpu.VMEM((128, Ch), jnp.float32),
            pltpu.VMEM((16, Ch), jnp.float32),
            pltpu.SemaphoreType.DMA,
            pltpu.SemaphoreType.DMA,
            pltpu.SemaphoreType.DMA,
            pltpu.SemaphoreType.DMA,
            pltpu.SemaphoreType.DMA,
            pltpu.SemaphoreType.DMA,
            pltpu.SemaphoreType.DMA,
            pltpu.SemaphoreType.DMA,
            pltpu.SemaphoreType.DMA,
        ],
    )
    def agg_kernel(
        ta_hbm, tb_hbm, sa_hbm, sb_hbm, src_hbm, dst_hbm, outa_hbm, outb_hbm,
        acc, is0, id0, is1, id1, idxs_t, idxd_t, rows0, rows1, rows_t,
        xsem0, ysem0, gsem0, ssem0, xsem1, ysem1, gsem1, ssem1, tsem,
    ):
        c = lax.axis_index("c")
        s = lax.axis_index("s")
        rbase = s * rows_per_tile

        def run(t_hbm, seed_hbm, out_hbm):
            base = c * core_stride + s * per_tile
            slots = ((is0, id0, rows0, xsem0, ysem0, gsem0, ssem0),
                     (is1, id1, rows1, xsem1, ysem1, gsem1, ssem1))

            def issue_idx(k, sl):
                isb, idb, _, xsem, ysem, _, _ = sl
                off = base + k * 128
                pltpu.async_copy(src_hbm.at[pl.ds(off, 128)], isb, xsem)
                pltpu.async_copy(dst_hbm.at[pl.ds(off, 128)], idb, ysem)

            def wait_idx(sl):
                isb, idb, _, xsem, ysem, _, _ = sl
                pltpu.make_async_copy(src_hbm.at[pl.ds(base, 128)], isb, xsem).wait()
                pltpu.make_async_copy(dst_hbm.at[pl.ds(base, 128)], idb, ysem).wait()

            def issue_gather(sl):
                isb, _, rws, _, _, gsem, _ = sl
                pltpu.async_copy(t_hbm.at[isb], rws, gsem)

            def wait_gather(sl):
                isb, _, rws, _, _, gsem, _ = sl
                pltpu.make_async_copy(t_hbm.at[isb], rws, gsem).wait()

            def issue_scatter(sl):
                _, idb, rws, _, _, _, ssem = sl
                pltpu.async_copy(rws, acc.at[idb], ssem, add=True)

            def wait_scatter(sl):
                _, idb, rws, _, _, _, ssem = sl
                pltpu.make_async_copy(rws, acc.at[idb], ssem).wait()

            issue_idx(0, slots[0])
            issue_idx(1, slots[1])
            pltpu.sync_copy(
                seed_hbm.at[pl.ds(rbase, rows_per_tile)],
                acc.at[pl.ds(rbase, rows_per_tile)],
            )
            wait_idx(slots[0])
            issue_gather(slots[0])
            plsc.subcore_barrier()

            def pair_body(g, carry):
                k0 = 2 * g
                s0, s1 = slots
                wait_gather(s0)
                issue_scatter(s0)
                wait_idx(s1)
                issue_gather(s1)
                wait_scatter(s0)

                @pl.when(k0 + 2 < n_chunks)
                def _():
                    issue_idx(k0 + 2, s0)

                wait_gather(s1)
                issue_scatter(s1)

                @pl.when(k0 + 2 < n_chunks)
                def _():
                    wait_idx(s0)
                    issue_gather(s0)

                wait_scatter(s1)

                @pl.when(k0 + 3 < n_chunks)
                def _():
                    issue_idx(k0 + 3, s1)

                return carry

            lax.fori_loop(0, n_chunks // 2, pair_body, 0)

            if n_chunks % 2:
                wait_gather(slots[0])
                issue_scatter(slots[0])
                wait_scatter(slots[0])

            if rem:
                off = base + per_tile - 16
                pltpu.sync_copy(src_hbm.at[pl.ds(off, 16)], idxs_t)
                pltpu.sync_copy(dst_hbm.at[pl.ds(off, 16)], idxd_t)
                if dup:
                    lane = lax.broadcasted_iota(jnp.int32, (16,), 0)
                    idxd_t[...] = jnp.where(lane < dup, 10000 + lane, idxd_t[...])
                pltpu.async_copy(t_hbm.at[idxs_t], rows_t, tsem).wait()
                pltpu.sync_copy(rows_t, acc.at[idxd_t], add=True)

            plsc.subcore_barrier()
            pltpu.sync_copy(
                acc.at[pl.ds(rbase, rows_per_tile)],
                out_hbm.at[pl.ds(rbase, rows_per_tile)],
            )

        @pl.when(c == 0)
        def _():
            run(ta_hbm, sa_hbm, outa_hbm)

        @pl.when(c == 1)
        def _():
            run(tb_hbm, sb_hbm, outb_hbm)

    return agg_kernel(t_a, t_b, seed_a, seed_b, src, dst)


def _tc_xw(x, W_enc):
    N, IN = x.shape
    HID = W_enc.shape[1]
    BM = 2000

    def body(x_ref, w_ref, o_ref):
        o_ref[...] = jnp.dot(x_ref[...], w_ref[...], preferred_element_type=jnp.float32)

    return pl.pallas_call(
        body,
        grid=(N // BM,),
        in_specs=[
            pl.BlockSpec((BM, IN), lambda m: (m, 0)),
            pl.BlockSpec((IN, HID), lambda m: (0, 0)),
        ],
        out_specs=pl.BlockSpec((BM, HID), lambda m: (m, 0)),
        out_shape=jax.ShapeDtypeStruct((N_PAD, HID), jnp.float32),
    )(x, W_enc)


def _tc_scale(parts, xw):
    NW, NP = parts.shape
    HID = xw.shape[1]
    BM = 2048

    def body(p_ref, xw_ref, t_ref, d_ref):
        ssum = jnp.sum(p_ref[...], axis=0, keepdims=True)
        d = lax.rsqrt(1.0 + ssum)
        i = lax.broadcasted_iota(jnp.int32, (128, 128), 0)
        j = lax.broadcasted_iota(jnp.int32, (128, 128), 1)
        eye = (i == j).astype(jnp.float32)
        for q in range(BM // 128):
            dq = d[:, q * 128 : (q + 1) * 128]
            d_ref[pl.ds(q * 128, 128), :] = lax.dot_general(
                eye, dq, (((1,), (1,)), ((), ())), preferred_element_type=jnp.float32
            )
        t_ref[...] = xw_ref[...] * d_ref[...]

    return pl.pallas_call(
        body,
        grid=(NP // BM,),
        in_specs=[
            pl.BlockSpec((NW, BM), lambda m: (0, m)),
            pl.BlockSpec((BM, HID), lambda m: (m, 0)),
        ],
        out_specs=[
            pl.BlockSpec((BM, HID), lambda m: (m, 0)),
            pl.BlockSpec((BM, 1), lambda m: (m, 0)),
        ],
        out_shape=[
            jax.ShapeDtypeStruct((NP, HID), jnp.float32),
            jax.ShapeDtypeStruct((NP, 1), jnp.float32),
        ],
    )(parts, xw)


def _tc_mid(g_a, g_b, dinv, b_enc, W_dec):
    _, HID = g_a.shape
    N = 10000
    OUT = W_dec.shape[1]
    O2 = OUT // 2
    BM = 2000

    def body(ga_ref, gb_ref, d_ref, b_ref, w_ref, oa_ref, ob_ref):
        agg = ga_ref[...] + gb_ref[...]
        d = d_ref[...]
        h = jnp.maximum(agg * d + b_ref[...], 0.0)
        hw = jnp.dot(h, w_ref[...], preferred_element_type=jnp.float32)
        t = hw * d
        oa_ref[...] = t[:, :O2]
        ob_ref[...] = t[:, O2:]

    return pl.pallas_call(
        body,
        grid=(N // BM,),
        in_specs=[
            pl.BlockSpec((BM, HID), lambda m: (m, 0)),
            pl.BlockSpec((BM, HID), lambda m: (m, 0)),
            pl.BlockSpec((BM, 1), lambda m: (m, 0)),
            pl.BlockSpec((1, HID), lambda m: (0, 0)),
            pl.BlockSpec((HID, OUT), lambda m: (0, 0)),
        ],
        out_specs=[
            pl.BlockSpec((BM, O2), lambda m: (m, 0)),
            pl.BlockSpec((BM, O2), lambda m: (m, 0)),
        ],
        out_shape=[
            jax.ShapeDtypeStruct((N_PAD, O2), jnp.float32),
            jax.ShapeDtypeStruct((N_PAD, O2), jnp.float32),
        ],
    )(g_a, g_b, dinv, b_enc, W_dec)


def _tc_out(g_a, g_b, dinv, b_dec):
    _, O2 = g_a.shape
    N = 10000
    OUT = 2 * O2
    BM = 2000

    def body(ga_ref, gb_ref, d_ref, b_ref, o_ref):
        agg = jnp.concatenate([ga_ref[...], gb_ref[...]], axis=1)
        o_ref[...] = agg * d_ref[...] + b_ref[...]

    return pl.pallas_call(
        body,
        grid=(N // BM,),
        in_specs=[
            pl.BlockSpec((BM, O2), lambda m: (m, 0)),
            pl.BlockSpec((BM, O2), lambda m: (m, 0)),
            pl.BlockSpec((BM, 1), lambda m: (m, 0)),
            pl.BlockSpec((1, OUT), lambda m: (0, 0)),
        ],
        out_specs=pl.BlockSpec((BM, OUT), lambda m: (m, 0)),
        out_shape=jax.ShapeDtypeStruct((N, OUT), jnp.float32),
    )(g_a, g_b, dinv, b_dec)


def kernel(x, edge_index, W_enc, b_enc, W_dec, b_dec):
    src = edge_index[0].astype(jnp.int32)
    dst = edge_index[1].astype(jnp.int32)

    E = src.shape[0]

    parts = _sc_deg(dst)
    xw = _tc_xw(x, W_enc)
    t1, dinv = _tc_scale(parts, xw)
    zeros = jnp.zeros_like(t1)
    g1a, g1b = _sc_agg(t1, t1, t1, zeros, src, dst, E // 2, E // 2)
    t2a, t2b = _tc_mid(g1a, g1b, dinv, b_enc.reshape(1, -1), W_dec)
    g2a, g2b = _sc_agg(t2a, t2b, t2a, t2b, src, dst, E, 0)
    return _tc_out(g2a, g2b, dinv, b_dec.reshape(1, -1))


# speedup vs baseline: 20.0190x; 1.0524x over previous
import functools

import jax
import jax.numpy as jnp
from jax import lax
from jax.experimental import pallas as pl
from jax.experimental.pallas import tpu as pltpu
from jax.experimental.pallas import tpu_sc as plsc

NC = 2
NS = 16
LANES = 16

N_PAD = 10240


def _sc_mesh():
    return plsc.VectorSubcoreMesh(
        core_axis_name="c", subcore_axis_name="s", num_cores=NC, num_subcores=NS
    )


def _sc_deg(ei, dst_off):
    E = ei.shape[0] // 2
    NW = NC * NS
    per_tile = E // NW
    n_chunks = per_tile // 128
    tail = per_tile - n_chunks * 128

    @functools.partial(
        pl.kernel,
        out_type=jax.ShapeDtypeStruct((NW, N_PAD), jnp.float32),
        mesh=_sc_mesh(),
        scratch_types=[
            pltpu.VMEM((N_PAD,), jnp.float32),
            pltpu.VMEM((128,), jnp.int32),
            pltpu.VMEM((128,), jnp.int32),
            pltpu.SemaphoreType.DMA,
            pltpu.SemaphoreType.DMA,
        ],
        compiler_params=pltpu.CompilerParams(needs_layout_passes=False),
    )
    def deg_kernel(ei_hbm, out_hbm, degv, idx0, idx1, sem0, sem1):
        c = lax.axis_index("c")
        s = lax.axis_index("s")
        wid = s * NC + c
        base = dst_off + wid * per_tile
        slots = ((idx0, sem0), (idx1, sem1))

        def issue(k, sl):
            buf, sem = sl
            pltpu.async_copy(ei_hbm.at[pl.ds(base + k * 128, 128)], buf, sem)

        def wait(sl):
            buf, sem = sl
            pltpu.make_async_copy(ei_hbm.at[pl.ds(base, 128)], buf, sem).wait()

        issue(0, slots[0])
        issue(1, slots[1])

        zeros = jnp.zeros((LANES,), jnp.float32)

        def zbody(i, carry):
            degv[pl.ds(i * LANES, LANES)] = zeros
            return carry

        lax.fori_loop(0, N_PAD // LANES, zbody, 0)

        ones = jnp.ones((LANES,), jnp.float32)

        def count(buf):
            for j in range(128 // LANES):
                idx = buf[pl.ds(j * LANES, LANES)]
                plsc.addupdate_scatter(degv, [idx], ones)

        def pair_body(g, carry):
            k0 = 2 * g
            wait(slots[0])
            count(idx0)

            @pl.when(k0 + 2 < n_chunks)
            def _():
                issue(k0 + 2, slots[0])

            wait(slots[1])
            count(idx1)

            @pl.when(k0 + 3 < n_chunks)
            def _():
                issue(k0 + 3, slots[1])

            return carry

        lax.fori_loop(0, n_chunks // 2, pair_body, 0)

        if n_chunks % 2:
            wait(slots[0])
            count(idx0)

        if tail:
            pltpu.sync_copy(
                ei_hbm.at[pl.ds(base + per_tile - LANES, LANES)], idx1.at[pl.ds(0, LANES)]
            )
            idx = idx1[pl.ds(0, LANES)]
            mask = lax.broadcasted_iota(jnp.int32, (LANES,), 0) >= (LANES - tail)
            plsc.addupdate_scatter(degv, [idx], ones, mask=mask)

        pltpu.sync_copy(degv, out_hbm.at[wid])

    return deg_kernel(ei)


def _sc_agg(t_a, t_b, seed_a, seed_b, ei, edges_per_core, core_stride):
    N, Ch = t_a.shape
    E = ei.shape[0] // 2
    per_tile = edges_per_core // NS
    n_chunks = per_tile // 128
    rem = per_tile - n_chunks * 128
    dup = 16 - rem if rem else 0
    rows_per_tile = N // NS

    @functools.partial(
        pl.kernel,
        out_type=(
            jax.ShapeDtypeStruct((N, Ch), jnp.float32),
            jax.ShapeDtypeStruct((N, Ch), jnp.float32),
        ),
        mesh=_sc_mesh(),
        scratch_types=[
            pltpu.VMEM_SHARED((N, Ch), jnp.float32),
            pltpu.VMEM((128,), jnp.int32),
            pltpu.VMEM((128,), jnp.int32),
            pltpu.VMEM((128,), jnp.int32),
            pltpu.VMEM((128,), jnp.int32),
            pltpu.VMEM((16,), jnp.int32),
            pltpu.VMEM((16,), jnp.int32),
            pltpu.VMEM((128, Ch), jnp.float32),
            pltpu.VMEM((128, Ch), jnp.float32),
            pltpu.VMEM((16, Ch), jnp.float32),
            pltpu.SemaphoreType.DMA,
            pltpu.SemaphoreType.DMA,
            pltpu.SemaphoreType.DMA,
            pltpu.SemaphoreType.DMA,
            pltpu.SemaphoreType.DMA,
            pltpu.SemaphoreType.DMA,
            pltpu.SemaphoreType.DMA,
            pltpu.SemaphoreType.DMA,
            pltpu.SemaphoreType.DMA,
        ],
    )
    def agg_kernel(
        ta_hbm, tb_hbm, sa_hbm, sb_hbm, ei_hbm, outa_hbm, outb_hbm,
        acc, is0, is1, id0, id1, idxs_t, idxd_t,
        rows0, rows1, rows_t,
        xsem0, ysem0, gsem0, ssem0, xsem1, ysem1, gsem1, ssem1, tsem,
    ):
        c = lax.axis_index("c")
        s = lax.axis_index("s")
        rbase = s * rows_per_tile

        def run(t_hbm, seed_hbm, out_hbm):
            base = c * core_stride + s * per_tile
            slots = ((is0, id0, rows0, xsem0, ysem0, gsem0, ssem0),
                     (is1, id1, rows1, xsem1, ysem1, gsem1, ssem1))

            def issue_idx(k, sl):
                isb, idb, _, xsem, ysem, _, _ = sl
                off = base + k * 128
                pltpu.async_copy(ei_hbm.at[pl.ds(off, 128)], isb, xsem)
                pltpu.async_copy(ei_hbm.at[pl.ds(E + off, 128)], idb, ysem)

            def wait_idx(sl):
                isb, idb, _, xsem, ysem, _, _ = sl
                pltpu.make_async_copy(ei_hbm.at[pl.ds(base, 128)], isb, xsem).wait()
                pltpu.make_async_copy(ei_hbm.at[pl.ds(base, 128)], idb, ysem).wait()

            def issue_gather(sl):
                isb, _, rws, _, _, gsem, _ = sl
                pltpu.async_copy(t_hbm.at[isb], rws, gsem)

            def wait_gather(sl):
                isb, _, rws, _, _, gsem, _ = sl
                pltpu.make_async_copy(t_hbm.at[isb], rws, gsem).wait()

            def issue_scatter(sl):
                _, idb, rws, _, _, _, ssem = sl
                pltpu.async_copy(rws, acc.at[idb], ssem, add=True)

            def wait_scatter(sl):
                _, idb, rws, _, _, _, ssem = sl
                pltpu.make_async_copy(rws, acc.at[idb], ssem).wait()

            issue_idx(0, slots[0])
            issue_idx(1, slots[1])
            pltpu.sync_copy(
                seed_hbm.at[pl.ds(rbase, rows_per_tile)],
                acc.at[pl.ds(rbase, rows_per_tile)],
            )
            wait_idx(slots[0])
            issue_gather(slots[0])
            plsc.subcore_barrier()

            def pair_body(g, carry):
                k0 = 2 * g
                s0, s1 = slots
                wait_gather(s0)
                issue_scatter(s0)
                wait_idx(s1)
                issue_gather(s1)
                wait_scatter(s0)

                @pl.when(k0 + 2 < n_chunks)
                def _():
                    issue_idx(k0 + 2, s0)

                wait_gather(s1)
                issue_scatter(s1)

                @pl.when(k0 + 2 < n_chunks)
                def _():
                    wait_idx(s0)
                    issue_gather(s0)

                wait_scatter(s1)

                @pl.when(k0 + 3 < n_chunks)
                def _():
                    issue_idx(k0 + 3, s1)

                return carry

            lax.fori_loop(0, n_chunks // 2, pair_body, 0)

            if n_chunks % 2:
                wait_gather(slots[0])
                issue_scatter(slots[0])
                wait_scatter(slots[0])

            if rem:
                off = base + per_tile - 16
                pltpu.sync_copy(ei_hbm.at[pl.ds(off, 16)], idxs_t)
                pltpu.sync_copy(ei_hbm.at[pl.ds(E + off, 16)], idxd_t)
                if dup:
                    lane = lax.broadcasted_iota(jnp.int32, (16,), 0)
                    idxd_t[...] = jnp.where(lane < dup, 10000 + lane, idxd_t[...])
                pltpu.async_copy(t_hbm.at[idxs_t], rows_t, tsem).wait()
                pltpu.sync_copy(rows_t, acc.at[idxd_t], add=True)

            plsc.subcore_barrier()
            pltpu.sync_copy(
                acc.at[pl.ds(rbase, rows_per_tile)],
                out_hbm.at[pl.ds(rbase, rows_per_tile)],
            )

        @pl.when(c == 0)
        def _():
            run(ta_hbm, sa_hbm, outa_hbm)

        @pl.when(c == 1)
        def _():
            run(tb_hbm, sb_hbm, outb_hbm)

    return agg_kernel(t_a, t_b, seed_a, seed_b, ei)


def _tc_xw(x, W_enc):
    N, IN = x.shape
    HID = W_enc.shape[1]
    BM = 2000

    def body(x_ref, w_ref, o_ref):
        o_ref[...] = jnp.dot(x_ref[...], w_ref[...], preferred_element_type=jnp.float32)

    return pl.pallas_call(
        body,
        grid=(N // BM,),
        in_specs=[
            pl.BlockSpec((BM, IN), lambda m: (m, 0)),
            pl.BlockSpec((IN, HID), lambda m: (0, 0)),
        ],
        out_specs=pl.BlockSpec((BM, HID), lambda m: (m, 0)),
        out_shape=jax.ShapeDtypeStruct((N_PAD, HID), jnp.float32),
    )(x, W_enc)


def _tc_scale(parts, xw):
    NW, NP = parts.shape
    HID = xw.shape[1]
    BM = 2048

    def body(p_ref, xw_ref, t_ref, d_ref):
        ssum = jnp.sum(p_ref[...], axis=0, keepdims=True)
        d = lax.rsqrt(1.0 + ssum)
        i = lax.broadcasted_iota(jnp.int32, (128, 128), 0)
        j = lax.broadcasted_iota(jnp.int32, (128, 128), 1)
        eye = (i == j).astype(jnp.float32)
        for q in range(BM // 128):
            dq = d[:, q * 128 : (q + 1) * 128]
            d_ref[pl.ds(q * 128, 128), :] = lax.dot_general(
                eye, dq, (((1,), (1,)), ((), ())), preferred_element_type=jnp.float32
            )
        t_ref[...] = xw_ref[...] * d_ref[...]

    return pl.pallas_call(
        body,
        grid=(NP // BM,),
        in_specs=[
            pl.BlockSpec((NW, BM), lambda m: (0, m)),
            pl.BlockSpec((BM, HID), lambda m: (m, 0)),
        ],
        out_specs=[
            pl.BlockSpec((BM, HID), lambda m: (m, 0)),
            pl.BlockSpec((BM, 1), lambda m: (m, 0)),
        ],
        out_shape=[
            jax.ShapeDtypeStruct((NP, HID), jnp.float32),
            jax.ShapeDtypeStruct((NP, 1), jnp.float32),
        ],
    )(parts, xw)


def _tc_mid(g_a, g_b, dinv, b_enc, W_dec):
    _, HID = g_a.shape
    N = 10000
    OUT = W_dec.shape[1]
    O2 = OUT // 2
    BM = 2000

    def body(ga_ref, gb_ref, d_ref, b_ref, w_ref, oa_ref, ob_ref):
        agg = ga_ref[...] + gb_ref[...]
        d = d_ref[...]
        h = jnp.maximum(agg * d + b_ref[...], 0.0)
        hw = jnp.dot(h, w_ref[...], preferred_element_type=jnp.float32)
        t = hw * d
        oa_ref[...] = t[:, :O2]
        ob_ref[...] = t[:, O2:]

    return pl.pallas_call(
        body,
        grid=(N // BM,),
        in_specs=[
            pl.BlockSpec((BM, HID), lambda m: (m, 0)),
            pl.BlockSpec((BM, HID), lambda m: (m, 0)),
            pl.BlockSpec((BM, 1), lambda m: (m, 0)),
            pl.BlockSpec((1, HID), lambda m: (0, 0)),
            pl.BlockSpec((HID, OUT), lambda m: (0, 0)),
        ],
        out_specs=[
            pl.BlockSpec((BM, O2), lambda m: (m, 0)),
            pl.BlockSpec((BM, O2), lambda m: (m, 0)),
        ],
        out_shape=[
            jax.ShapeDtypeStruct((N_PAD, O2), jnp.float32),
            jax.ShapeDtypeStruct((N_PAD, O2), jnp.float32),
        ],
    )(g_a, g_b, dinv, b_enc, W_dec)


def _tc_out(g_a, g_b, dinv, b_dec):
    _, O2 = g_a.shape
    N = 10000
    OUT = 2 * O2
    BM = 2000

    def body(ga_ref, gb_ref, d_ref, b_ref, o_ref):
        agg = jnp.concatenate([ga_ref[...], gb_ref[...]], axis=1)
        o_ref[...] = agg * d_ref[...] + b_ref[...]

    return pl.pallas_call(
        body,
        grid=(N // BM,),
        in_specs=[
            pl.BlockSpec((BM, O2), lambda m: (m, 0)),
            pl.BlockSpec((BM, O2), lambda m: (m, 0)),
            pl.BlockSpec((BM, 1), lambda m: (m, 0)),
            pl.BlockSpec((1, OUT), lambda m: (0, 0)),
        ],
        out_specs=pl.BlockSpec((BM, OUT), lambda m: (m, 0)),
        out_shape=jax.ShapeDtypeStruct((N, OUT), jnp.float32),
    )(g_a, g_b, dinv, b_dec)


def kernel(x, edge_index, W_enc, b_enc, W_dec, b_dec):
    E = edge_index.shape[1]
    ei = edge_index.astype(jnp.int32).reshape(-1)

    parts = _sc_deg(ei, E)
    xw = _tc_xw(x, W_enc)
    t1, dinv = _tc_scale(parts, xw)
    zeros = jnp.zeros_like(t1)
    g1a, g1b = _sc_agg(t1, t1, t1, zeros, ei, E // 2, E // 2)
    t2a, t2b = _tc_mid(g1a, g1b, dinv, b_enc.reshape(1, -1), W_dec)
    g2a, g2b = _sc_agg(t2a, t2b, t2a, t2b, ei, E, 0)
    return _tc_out(g2a, g2b, dinv, b_dec.reshape(1, -1))
